# baseline (speedup 1.0000x reference)
import functools

import jax
import jax.numpy as jnp
from jax import lax
from jax.experimental import pallas as pl
from jax.experimental.pallas import tpu as pltpu
from jax.experimental.pallas import tpu_sc as plsc

N = 2097152
EPS = 1e-07
RATIO = 0.3

_NCORE = 2
_NSUB = 16
_NWORK = _NCORE * _NSUB
_PER_SUB = N // _NWORK
_CHUNK = 8192
_NCHUNK = _PER_SUB // _CHUNK
_VECS = _CHUNK // 16
_HIST = 16 * 512
_UNROLL = 4

_TOPBIT = -(2**31)
_MASK31 = 0x7FFFFFFF


def _loss_key_body(x_ref, z_ref, key_ref):
    x = x_ref[...]
    z = z_ref[...]
    probs = jnp.clip(jax.nn.sigmoid(x), EPS, 1.0 - EPS)
    pt = probs * z + (1.0 - probs) * (1.0 - z)
    bce = jnp.maximum(x, 0.0) - x * z + jnp.log1p(jnp.exp(-jnp.abs(x)))
    pos = z == 1.0
    one_m = 1.0 - pt
    focal = jnp.where(pos, one_m * one_m, one_m)
    alpha = jnp.where(pos, jnp.float32(1.0), jnp.float32(0.5))
    loss = alpha * focal * bce
    bits = lax.bitcast_convert_type(loss, jnp.int32)
    key_ref[...] = jnp.where(pos, bits | _TOPBIT, bits)


def _compute_keys(inputs, targets):
    blk = N // 8
    return pl.pallas_call(
        _loss_key_body,
        grid=(8,),
        in_specs=[
            pl.BlockSpec((blk,), lambda i: (i,)),
            pl.BlockSpec((blk,), lambda i: (i,)),
        ],
        out_specs=pl.BlockSpec((blk,), lambda i: (i,)),
        out_shape=jax.ShapeDtypeStruct((N,), jnp.int32),
    )(inputs, targets)


def _lane_i(lane, v, j):
    return jnp.sum(jnp.where(lane == j, v, jnp.int32(0)))


def _lane_f(lane, v, j):
    return jnp.sum(jnp.where(lane == j, v, jnp.float32(0.0)))


def _revcumsum(v):
    return lax.rev(jnp.cumsum(lax.rev(v, (0,))), (0,))


def _select_level(l, lane, red_c, red_s, st):
    P, c_above, s_above, kk, alive = [list(x) for x in st]
    shift = 24 - 8 * l
    zeros_i = jnp.zeros((16,), jnp.int32)
    zeros_f = jnp.zeros((16,), jnp.float32)
    for cls in range(2):
        base = cls * 256
        chunkV = zeros_i
        chunkS = zeros_f
        for ci in range(16):
            vc = red_c[pl.ds(base + ci * 16, 16)]
            vs = red_s[pl.ds(base + ci * 16, 16)]
            chunkV = jnp.where(lane == ci, jnp.sum(vc), chunkV)
            chunkS = jnp.where(lane == ci, jnp.sum(vs), chunkS)
        if l == 0:
            n_cls = jnp.sum(chunkV)
            alive[cls] = (n_cls > 0).astype(jnp.int32)
            kf = n_cls.astype(jnp.float32) * jnp.float32(RATIO)
            ki = kf.astype(jnp.int32)
            ki = ki - (ki.astype(jnp.float32) > kf).astype(jnp.int32)
            kk[cls] = jnp.maximum(jnp.int32(1), ki)
        r = kk[cls] - c_above[cls]
        SCi = _revcumsum(chunkV)
        SSi = _revcumsum(chunkS)
        I = jnp.maximum(jnp.max(plsc.all_reduce_population_count(SCi >= r)) - 1, 0)
        A_c = _lane_i(lane, SCi - chunkV, I)
        A_s = _lane_f(lane, SSi - chunkS, I)
        c16 = red_c[pl.ds(base + I * 16, 16)]
        s16 = red_s[pl.ds(base + I * 16, 16)]
        W = _revcumsum(c16)
        Ws = _revcumsum(s16)
        jj = jnp.maximum(jnp.max(plsc.all_reduce_population_count((A_c + W) >= r)) - 1, 0)
        B = I * 16 + jj
        cn = c_above[cls] + A_c + _lane_i(lane, W - c16, jj)
        sn = s_above[cls] + A_s + _lane_f(lane, Ws - s16, jj)
        pn = P[cls] | lax.shift_left(B, jnp.int32(shift))
        ok = alive[cls] > 0
        c_above[cls] = jnp.where(ok, cn, c_above[cls])
        s_above[cls] = jnp.where(ok, sn, s_above[cls])
        P[cls] = jnp.where(ok, pn, P[cls])
    return P, c_above, s_above, kk, alive


def _merge_prev(prev_c, prev_s, mp_c, mp_s, red_c, red_s):
    pltpu.sync_copy(prev_c, mp_c)
    pltpu.sync_copy(prev_s, mp_s)

    def _m(j, _):
        red_c[pl.ds(j * 16, 16)] = (mp_c[0, pl.ds(j * 16, 16)]
                                    + mp_c[1, pl.ds(j * 16, 16)])
        red_s[pl.ds(j * 16, 16)] = (mp_s[0, pl.ds(j * 16, 16)]
                                    + mp_s[1, pl.ds(j * 16, 16)])
        return 0

    lax.fori_loop(0, 32, _m, 0)


def _unpack_state(lane, stv_i, stv_f):
    vi = stv_i[...]
    vf = stv_f[...]
    P = [_lane_i(lane, vi, 0), _lane_i(lane, vi, 1)]
    c_above = [_lane_i(lane, vi, 2), _lane_i(lane, vi, 3)]
    kk = [_lane_i(lane, vi, 4), _lane_i(lane, vi, 5)]
    alive = [_lane_i(lane, vi, 6), _lane_i(lane, vi, 7)]
    s_above = [_lane_f(lane, vf, 0), _lane_f(lane, vf, 1)]
    return P, c_above, s_above, kk, alive


def _pack_state(lane, st):
    P, c_above, s_above, kk, alive = st
    vals_i = [P[0], P[1], c_above[0], c_above[1], kk[0], kk[1], alive[0], alive[1]]
    vi = jnp.zeros((16,), jnp.int32)
    for j, v in enumerate(vals_i):
        vi = jnp.where(lane == j, v, vi)
    vf = jnp.zeros((16,), jnp.float32)
    vf = jnp.where(lane == 0, s_above[0], vf)
    vf = jnp.where(lane == 1, s_above[1], vf)
    return vi, vf


def _init_state():
    z = jnp.zeros((), jnp.int32)
    zf = jnp.zeros((), jnp.float32)
    return ([z, z + _TOPBIT], [z, z], [zf, zf], [z, z], [z, z])


def _scan_body(l, keys_hbm, prev_c, prev_s, st_i_in, st_f_in,
               out_c, out_s, st_i_out, st_f_out,
               buf0, buf1, hist_c, hist_s, red_c, red_s, mp_c, mp_s, mc, ms,
               stv_i, stv_f, stage_c, stage_s, sem0, sem1):
    sid = lax.axis_index("s")
    cid = lax.axis_index("c")
    wid = cid * _NSUB + sid
    lane = lax.iota(jnp.int32, 16)
    lane_base = lane * jnp.int32(512)
    ones_i = jnp.ones((16,), jnp.int32)
    zeros_i = jnp.zeros((16,), jnp.int32)
    zeros_f = jnp.zeros((16,), jnp.float32)
    ubase = [lane_base + u * _HIST for u in range(_UNROLL)]

    if l == 0:
        st = _init_state()
    else:
        _merge_prev(prev_c, prev_s, mp_c, mp_s, red_c, red_s)
        if l == 1:
            st = _init_state()
        else:
            pltpu.sync_copy(st_i_in.at[0], stv_i)
            pltpu.sync_copy(st_f_in.at[0], stv_f)
            st = _unpack_state(lane, stv_i, stv_f)
        st = _select_level(l - 1, lane, red_c, red_s, st)
        vi, vf = _pack_state(lane, st)
        stv_i[...] = vi
        stv_f[...] = vf

    P = st[0]
    shift = 24 - 8 * l
    mask_hi = _TOPBIT if l == 0 else -(1 << (32 - 8 * l))
    shift_v = jnp.full((16,), shift, jnp.int32)

    def _zero(i, _):
        hist_c[pl.ds(i * 16, 16)] = zeros_i
        hist_s[pl.ds(i * 16, 16)] = zeros_f
        return 0

    lax.fori_loop(0, _UNROLL * _HIST // 16, _zero, 0)

    Pn, Pp = P[0], P[1]

    def _scan_buf(buf):
        def _scan(i, _):
            vo = i * (16 * _UNROLL)
            idxs, losses, ms_ = [], [], []
            for u in range(_UNROLL):
                x = buf[pl.ds(vo + u * 16, 16)]
                mn = ((x ^ Pn) & mask_hi) == 0
                mp = ((x ^ Pp) & mask_hi) == 0
                b = lax.shift_right_logical(x, shift_v) & jnp.int32(0xFF)
                idxs.append(ubase[u] + b
                            + jnp.where(mp, jnp.int32(256), jnp.int32(0)))
                ms_.append(mn | mp)
                losses.append(plsc.bitcast(x & _MASK31, jnp.float32))
            for u in range(_UNROLL):
                plsc.addupdate_scatter(hist_c, [idxs[u]], ones_i, mask=ms_[u])
                plsc.addupdate_scatter(hist_s, [idxs[u]], losses[u], mask=ms_[u])
            return 0

        lax.fori_loop(0, _VECS // _UNROLL, _scan, 0)

    def _chunk_slice(c):
        return keys_hbm.at[pl.ds(wid * _PER_SUB + c * _CHUNK, _CHUNK)]

    pltpu.async_copy(_chunk_slice(0), buf0, sem0)

    def _dbl(j, _):
        pltpu.async_copy(_chunk_slice(2 * j + 1), buf1, sem1)
        pltpu.make_async_copy(_chunk_slice(0), buf0, sem0).wait()
        _scan_buf(buf0)
        pltpu.async_copy(_chunk_slice(jnp.minimum(2 * j + 2, _NCHUNK - 1)),
                         buf0, sem0)
        pltpu.make_async_copy(_chunk_slice(0), buf1, sem1).wait()
        _scan_buf(buf1)
        return 0

    lax.fori_loop(0, _NCHUNK // 2, _dbl, 0)
    pltpu.make_async_copy(_chunk_slice(0), buf0, sem0).wait()

    def _fold(i, _):
        o = i * 16
        hist_c[pl.ds(o, 16)] = (hist_c[pl.ds(o, 16)] + hist_c[pl.ds(o + _HIST, 16)]
                                + hist_c[pl.ds(o + 2 * _HIST, 16)]
                                + hist_c[pl.ds(o + 3 * _HIST, 16)])
        hist_s[pl.ds(o, 16)] = (hist_s[pl.ds(o, 16)] + hist_s[pl.ds(o + _HIST, 16)]
                                + hist_s[pl.ds(o + 2 * _HIST, 16)]
                                + hist_s[pl.ds(o + 3 * _HIST, 16)])
        return 0

    lax.fori_loop(0, _HIST // 16, _fold, 0)

    def _lred(j, _):
        def _acc(ln, carry):
            ac, asum = carry
            off = ln * jnp.int32(512) + j * 16
            return ac + hist_c[pl.ds(off, 16)], asum + hist_s[pl.ds(off, 16)]

        ac, asum = lax.fori_loop(0, 16, _acc, (zeros_i, zeros_f))
        red_c[pl.ds(j * 16, 16)] = ac
        red_s[pl.ds(j * 16, 16)] = asum
        return 0

    lax.fori_loop(0, 32, _lred, 0)

    pltpu.sync_copy(red_c, stage_c.at[sid])
    pltpu.sync_copy(red_s, stage_s.at[sid])
    plsc.subcore_barrier()

    @pl.when(sid == 0)
    def _():
        def _gagg(j, _):
            def _acc(s, carry):
                ac, asum = carry
                return (ac + mc[s, pl.ds(j * 16, 16)],
                        asum + ms[s, pl.ds(j * 16, 16)])

            ac, asum = lax.fori_loop(0, 16, _acc, (zeros_i, zeros_f))
            red_c[pl.ds(j * 16, 16)] = ac
            red_s[pl.ds(j * 16, 16)] = asum
            return 0

        pltpu.sync_copy(stage_c, mc)
        pltpu.sync_copy(stage_s, ms)
        lax.fori_loop(0, 32, _gagg, 0)
        pltpu.sync_copy(red_c, out_c.at[cid])
        pltpu.sync_copy(red_s, out_s.at[cid])
        if l > 0:
            pltpu.sync_copy(stv_i, st_i_out.at[cid])
            pltpu.sync_copy(stv_f, st_f_out.at[cid])


def _final_body(prev_c, prev_s, st_i_in, st_f_in, out_hbm,
                red_c, red_s, mp_c, mp_s, stv_i, stv_f, outv):
    sid = lax.axis_index("s")
    cid = lax.axis_index("c")
    lane = lax.iota(jnp.int32, 16)

    @pl.when((sid == 0) & (cid == 0))
    def _():
        _merge_prev(prev_c, prev_s, mp_c, mp_s, red_c, red_s)
        pltpu.sync_copy(st_i_in.at[0], stv_i)
        pltpu.sync_copy(st_f_in.at[0], stv_f)
        st = _unpack_state(lane, stv_i, stv_f)
        P, c_above, s_above, kk, alive = _select_level(3, lane, red_c, red_s, st)
        num = jnp.zeros((16,), jnp.float32)
        den = jnp.zeros((), jnp.float32)
        for cls in range(2):
            t_bits = jnp.zeros((16,), jnp.int32) + (P[cls] & _MASK31)
            t_f = plsc.bitcast(t_bits, jnp.float32)
            contrib = s_above[cls] + (kk[cls] - c_above[cls]).astype(jnp.float32) * t_f
            af = alive[cls].astype(jnp.float32)
            num = num + af * contrib
            den = den + af * kk[cls].astype(jnp.float32)
        outv[...] = num / den
        pltpu.sync_copy(outv, out_hbm)


def _sc_select(keys):
    mesh = plsc.VectorSubcoreMesh(core_axis_name="c", subcore_axis_name="s",
                                  num_cores=_NCORE)
    params = pltpu.CompilerParams(needs_layout_passes=False)
    hist_out = (jax.ShapeDtypeStruct((_NCORE, 512), jnp.int32),
                jax.ShapeDtypeStruct((_NCORE, 512), jnp.float32))
    st_out = (jax.ShapeDtypeStruct((_NCORE, 16), jnp.int32),
              jax.ShapeDtypeStruct((_NCORE, 16), jnp.float32))
    scan_scratch = [
        pltpu.VMEM((_CHUNK,), jnp.int32),
        pltpu.VMEM((_CHUNK,), jnp.int32),
        pltpu.VMEM((_UNROLL * _HIST,), jnp.int32),
        pltpu.VMEM((_UNROLL * _HIST,), jnp.float32),
        pltpu.VMEM((512,), jnp.int32),
        pltpu.VMEM((512,), jnp.float32),
        pltpu.VMEM((2, 512), jnp.int32),
        pltpu.VMEM((2, 512), jnp.float32),
        pltpu.VMEM((16, 512), jnp.int32),
        pltpu.VMEM((16, 512), jnp.float32),
        pltpu.VMEM((16,), jnp.int32),
        pltpu.VMEM((16,), jnp.float32),
        pltpu.VMEM_SHARED((16, 512), jnp.int32),
        pltpu.VMEM_SHARED((16, 512), jnp.float32),
        pltpu.SemaphoreType.DMA,
        pltpu.SemaphoreType.DMA,
    ]

    zc = jnp.zeros((_NCORE, 512), jnp.int32)
    zs = jnp.zeros((_NCORE, 512), jnp.float32)
    zi = jnp.zeros((_NCORE, 16), jnp.int32)
    zf = jnp.zeros((_NCORE, 16), jnp.float32)

    hc, hs = None, None
    sti, stf = zi, zf
    for l in range(4):
        f = pl.kernel(
            functools.partial(_scan_body, l),
            out_type=hist_out + st_out,
            mesh=mesh,
            compiler_params=params,
            scratch_types=scan_scratch,
        )
        hc, hs, sti_n, stf_n = f(keys,
                                 zc if hc is None else hc,
                                 zs if hs is None else hs,
                                 sti, stf)
        if l > 0:
            sti, stf = sti_n, stf_n

    f = pl.kernel(
        _final_body,
        out_type=jax.ShapeDtypeStruct((16,), jnp.float32),
        mesh=mesh,
        compiler_params=params,
        scratch_types=[
            pltpu.VMEM((512,), jnp.int32),
            pltpu.VMEM((512,), jnp.float32),
            pltpu.VMEM((2, 512), jnp.int32),
            pltpu.VMEM((2, 512), jnp.float32),
            pltpu.VMEM((16,), jnp.int32),
            pltpu.VMEM((16,), jnp.float32),
            pltpu.VMEM((16,), jnp.float32),
        ],
    )
    return f(hc, hs, sti, stf)


def kernel(inputs, targets):
    keys = _compute_keys(inputs, targets)
    out = _sc_select(keys)
    return out[0]


# speedup vs baseline: 1.1133x; 1.1133x over previous
import functools

import jax
import jax.numpy as jnp
from jax import lax
from jax.experimental import pallas as pl
from jax.experimental.pallas import tpu as pltpu
from jax.experimental.pallas import tpu_sc as plsc

N = 2097152
EPS = 1e-07
RATIO = 0.3

_NCORE = 2
_NSUB = 16
_NWORK = _NCORE * _NSUB
_PER_SUB = N // _NWORK
_CHUNK = 8192
_NCHUNK = _PER_SUB // _CHUNK
_VECS = _CHUNK // 16
_HIST = 16 * 512
_UNROLL = 4
_COPIES = 2

_TOPBIT = -(2**31)
_MASK31 = 0x7FFFFFFF


def _loss_key_body(x_ref, z_ref, key_ref):
    x = x_ref[...]
    z = z_ref[...]
    probs = jnp.clip(jax.nn.sigmoid(x), EPS, 1.0 - EPS)
    pt = probs * z + (1.0 - probs) * (1.0 - z)
    bce = jnp.maximum(x, 0.0) - x * z + jnp.log1p(jnp.exp(-jnp.abs(x)))
    pos = z == 1.0
    one_m = 1.0 - pt
    focal = jnp.where(pos, one_m * one_m, one_m)
    alpha = jnp.where(pos, jnp.float32(1.0), jnp.float32(0.5))
    loss = alpha * focal * bce
    bits = lax.bitcast_convert_type(loss, jnp.int32)
    key_ref[...] = jnp.where(pos, bits | _TOPBIT, bits)


def _compute_keys(inputs, targets):
    blk = N // 8
    return pl.pallas_call(
        _loss_key_body,
        grid=(8,),
        in_specs=[
            pl.BlockSpec((blk,), lambda i: (i,)),
            pl.BlockSpec((blk,), lambda i: (i,)),
        ],
        out_specs=pl.BlockSpec((blk,), lambda i: (i,)),
        out_shape=jax.ShapeDtypeStruct((N,), jnp.int32),
    )(inputs, targets)


def _lane_i(lane, v, j):
    return jnp.sum(jnp.where(lane == j, v, jnp.int32(0)))


def _lane_f(lane, v, j):
    return jnp.sum(jnp.where(lane == j, v, jnp.float32(0.0)))


def _revcumsum(v):
    return lax.rev(jnp.cumsum(lax.rev(v, (0,))), (0,))


def _select_level(l, lane, red_c, red_s, st):
    P, c_above, s_above, kk, alive = [list(x) for x in st]
    shift = 24 - 8 * l
    zeros_i = jnp.zeros((16,), jnp.int32)
    zeros_f = jnp.zeros((16,), jnp.float32)
    for cls in range(2):
        base = cls * 256
        chunkV = zeros_i
        chunkS = zeros_f
        for ci in range(16):
            vc = red_c[pl.ds(base + ci * 16, 16)]
            vs = red_s[pl.ds(base + ci * 16, 16)]
            chunkV = jnp.where(lane == ci, jnp.sum(vc), chunkV)
            chunkS = jnp.where(lane == ci, jnp.sum(vs), chunkS)
        if l == 0:
            n_cls = jnp.sum(chunkV)
            alive[cls] = (n_cls > 0).astype(jnp.int32)
            kf = n_cls.astype(jnp.float32) * jnp.float32(RATIO)
            ki = kf.astype(jnp.int32)
            ki = ki - (ki.astype(jnp.float32) > kf).astype(jnp.int32)
            kk[cls] = jnp.maximum(jnp.int32(1), ki)
        r = kk[cls] - c_above[cls]
        SCi = _revcumsum(chunkV)
        SSi = _revcumsum(chunkS)
        I = jnp.maximum(jnp.max(plsc.all_reduce_population_count(SCi >= r)) - 1, 0)
        A_c = _lane_i(lane, SCi - chunkV, I)
        A_s = _lane_f(lane, SSi - chunkS, I)
        c16 = red_c[pl.ds(base + I * 16, 16)]
        s16 = red_s[pl.ds(base + I * 16, 16)]
        W = _revcumsum(c16)
        Ws = _revcumsum(s16)
        jj = jnp.maximum(jnp.max(plsc.all_reduce_population_count((A_c + W) >= r)) - 1, 0)
        B = I * 16 + jj
        cn = c_above[cls] + A_c + _lane_i(lane, W - c16, jj)
        sn = s_above[cls] + A_s + _lane_f(lane, Ws - s16, jj)
        pn = P[cls] | lax.shift_left(B, jnp.int32(shift))
        ok = alive[cls] > 0
        c_above[cls] = jnp.where(ok, cn, c_above[cls])
        s_above[cls] = jnp.where(ok, sn, s_above[cls])
        P[cls] = jnp.where(ok, pn, P[cls])
    return P, c_above, s_above, kk, alive


def _merge_prev(prev_c, prev_s, mp_c, mp_s, red_c, red_s):
    pltpu.sync_copy(prev_c, mp_c)
    pltpu.sync_copy(prev_s, mp_s)

    def _m(j, _):
        red_c[pl.ds(j * 16, 16)] = (mp_c[0, pl.ds(j * 16, 16)]
                                    + mp_c[1, pl.ds(j * 16, 16)])
        red_s[pl.ds(j * 16, 16)] = (mp_s[0, pl.ds(j * 16, 16)]
                                    + mp_s[1, pl.ds(j * 16, 16)])
        return 0

    lax.fori_loop(0, 32, _m, 0)


def _unpack_state(lane, stv_i, stv_f):
    vi = stv_i[...]
    vf = stv_f[...]
    P = [_lane_i(lane, vi, 0), _lane_i(lane, vi, 1)]
    c_above = [_lane_i(lane, vi, 2), _lane_i(lane, vi, 3)]
    kk = [_lane_i(lane, vi, 4), _lane_i(lane, vi, 5)]
    alive = [_lane_i(lane, vi, 6), _lane_i(lane, vi, 7)]
    s_above = [_lane_f(lane, vf, 0), _lane_f(lane, vf, 1)]
    return P, c_above, s_above, kk, alive


def _pack_state(lane, st):
    P, c_above, s_above, kk, alive = st
    vals_i = [P[0], P[1], c_above[0], c_above[1], kk[0], kk[1], alive[0], alive[1]]
    vi = jnp.zeros((16,), jnp.int32)
    for j, v in enumerate(vals_i):
        vi = jnp.where(lane == j, v, vi)
    vf = jnp.zeros((16,), jnp.float32)
    vf = jnp.where(lane == 0, s_above[0], vf)
    vf = jnp.where(lane == 1, s_above[1], vf)
    return vi, vf


def _init_state():
    z = jnp.zeros((), jnp.int32)
    zf = jnp.zeros((), jnp.float32)
    return ([z, z + _TOPBIT], [z, z], [zf, zf], [z, z], [z, z])


def _scan_body(l, keys_hbm, prev_c, prev_s, st_i_in, st_f_in,
               out_c, out_s, st_i_out, st_f_out,
               buf0, buf1, hist_c, hist_s, red_c, red_s, mp_c, mp_s, mc, ms,
               stv_i, stv_f, stage_c, stage_s, sem0, sem1):
    sid = lax.axis_index("s")
    cid = lax.axis_index("c")
    wid = cid * _NSUB + sid
    lane = lax.iota(jnp.int32, 16)
    lane_base = lane * jnp.int32(512)
    ones_i = jnp.ones((16,), jnp.int32)
    zeros_i = jnp.zeros((16,), jnp.int32)
    zeros_f = jnp.zeros((16,), jnp.float32)
    ubase = [lane_base + (u % _COPIES) * _HIST for u in range(_UNROLL)]

    if l == 0:
        st = _init_state()
    else:
        _merge_prev(prev_c, prev_s, mp_c, mp_s, red_c, red_s)
        if l == 1:
            st = _init_state()
        else:
            pltpu.sync_copy(st_i_in.at[0], stv_i)
            pltpu.sync_copy(st_f_in.at[0], stv_f)
            st = _unpack_state(lane, stv_i, stv_f)
        st = _select_level(l - 1, lane, red_c, red_s, st)
        vi, vf = _pack_state(lane, st)
        stv_i[...] = vi
        stv_f[...] = vf

    P = st[0]
    shift = 24 - 8 * l
    mask_hi = _TOPBIT if l == 0 else -(1 << (32 - 8 * l))
    shift_v = jnp.full((16,), shift, jnp.int32)

    def _zero(i, _):
        hist_c[pl.ds(i * 16, 16)] = zeros_i
        hist_s[pl.ds(i * 16, 16)] = zeros_f
        return 0

    lax.fori_loop(0, _COPIES * _HIST // 16, _zero, 0)

    Pn, Pp = P[0], P[1]

    def _scan_buf(buf):
        def _scan(i, _):
            vo = i * (16 * _UNROLL)
            idxs, losses, ms_ = [], [], []
            for u in range(_UNROLL):
                x = buf[pl.ds(vo + u * 16, 16)]
                mn = ((x ^ Pn) & mask_hi) == 0
                mp = ((x ^ Pp) & mask_hi) == 0
                b = lax.shift_right_logical(x, shift_v) & jnp.int32(0xFF)
                idxs.append(ubase[u] + b
                            + jnp.where(mp, jnp.int32(256), jnp.int32(0)))
                ms_.append(mn | mp)
                losses.append(plsc.bitcast(x & _MASK31, jnp.float32))
            for u in range(_UNROLL):
                plsc.addupdate_scatter(hist_c, [idxs[u]], ones_i, mask=ms_[u])
                plsc.addupdate_scatter(hist_s, [idxs[u]], losses[u], mask=ms_[u])
            return 0

        lax.fori_loop(0, _VECS // _UNROLL, _scan, 0)

    def _chunk_slice(c):
        return keys_hbm.at[pl.ds(wid * _PER_SUB + c * _CHUNK, _CHUNK)]

    pltpu.async_copy(_chunk_slice(0), buf0, sem0)

    def _dbl(j, _):
        pltpu.async_copy(_chunk_slice(2 * j + 1), buf1, sem1)
        pltpu.make_async_copy(_chunk_slice(0), buf0, sem0).wait()
        _scan_buf(buf0)
        pltpu.async_copy(_chunk_slice(jnp.minimum(2 * j + 2, _NCHUNK - 1)),
                         buf0, sem0)
        pltpu.make_async_copy(_chunk_slice(0), buf1, sem1).wait()
        _scan_buf(buf1)
        return 0

    lax.fori_loop(0, _NCHUNK // 2, _dbl, 0)
    pltpu.make_async_copy(_chunk_slice(0), buf0, sem0).wait()

    def _fold(i, _):
        o = i * 16
        hist_c[pl.ds(o, 16)] = sum(hist_c[pl.ds(o + u * _HIST, 16)]
                                   for u in range(1, _COPIES)) + hist_c[pl.ds(o, 16)]
        hist_s[pl.ds(o, 16)] = sum(hist_s[pl.ds(o + u * _HIST, 16)]
                                   for u in range(1, _COPIES)) + hist_s[pl.ds(o, 16)]
        return 0

    lax.fori_loop(0, _HIST // 16, _fold, 0)

    def _lred(j, _):
        def _acc(ln, carry):
            ac, asum = carry
            off = ln * jnp.int32(512) + j * 16
            return ac + hist_c[pl.ds(off, 16)], asum + hist_s[pl.ds(off, 16)]

        ac, asum = lax.fori_loop(0, 16, _acc, (zeros_i, zeros_f))
        red_c[pl.ds(j * 16, 16)] = ac
        red_s[pl.ds(j * 16, 16)] = asum
        return 0

    lax.fori_loop(0, 32, _lred, 0)

    pltpu.sync_copy(red_c, stage_c.at[sid])
    pltpu.sync_copy(red_s, stage_s.at[sid])
    plsc.subcore_barrier()

    @pl.when(sid == 0)
    def _():
        def _gagg(j, _):
            def _acc(s, carry):
                ac, asum = carry
                return (ac + mc[s, pl.ds(j * 16, 16)],
                        asum + ms[s, pl.ds(j * 16, 16)])

            ac, asum = lax.fori_loop(0, 16, _acc, (zeros_i, zeros_f))
            red_c[pl.ds(j * 16, 16)] = ac
            red_s[pl.ds(j * 16, 16)] = asum
            return 0

        pltpu.sync_copy(stage_c, mc)
        pltpu.sync_copy(stage_s, ms)
        lax.fori_loop(0, 32, _gagg, 0)
        pltpu.sync_copy(red_c, out_c.at[cid])
        pltpu.sync_copy(red_s, out_s.at[cid])
        if l > 0:
            pltpu.sync_copy(stv_i, st_i_out.at[cid])
            pltpu.sync_copy(stv_f, st_f_out.at[cid])


def _final_body(prev_c, prev_s, st_i_in, st_f_in, out_hbm,
                red_c, red_s, mp_c, mp_s, stv_i, stv_f, outv):
    sid = lax.axis_index("s")
    cid = lax.axis_index("c")
    lane = lax.iota(jnp.int32, 16)

    @pl.when((sid == 0) & (cid == 0))
    def _():
        _merge_prev(prev_c, prev_s, mp_c, mp_s, red_c, red_s)
        pltpu.sync_copy(st_i_in.at[0], stv_i)
        pltpu.sync_copy(st_f_in.at[0], stv_f)
        st = _unpack_state(lane, stv_i, stv_f)
        P, c_above, s_above, kk, alive = _select_level(3, lane, red_c, red_s, st)
        num = jnp.zeros((16,), jnp.float32)
        den = jnp.zeros((), jnp.float32)
        for cls in range(2):
            t_bits = jnp.zeros((16,), jnp.int32) + (P[cls] & _MASK31)
            t_f = plsc.bitcast(t_bits, jnp.float32)
            contrib = s_above[cls] + (kk[cls] - c_above[cls]).astype(jnp.float32) * t_f
            af = alive[cls].astype(jnp.float32)
            num = num + af * contrib
            den = den + af * kk[cls].astype(jnp.float32)
        outv[...] = num / den
        pltpu.sync_copy(outv, out_hbm)


def _sc_select(keys):
    mesh = plsc.VectorSubcoreMesh(core_axis_name="c", subcore_axis_name="s",
                                  num_cores=_NCORE)
    params = pltpu.CompilerParams(needs_layout_passes=False)
    hist_out = (jax.ShapeDtypeStruct((_NCORE, 512), jnp.int32),
                jax.ShapeDtypeStruct((_NCORE, 512), jnp.float32))
    st_out = (jax.ShapeDtypeStruct((_NCORE, 16), jnp.int32),
              jax.ShapeDtypeStruct((_NCORE, 16), jnp.float32))
    scan_scratch = [
        pltpu.VMEM((_CHUNK,), jnp.int32),
        pltpu.VMEM((_CHUNK,), jnp.int32),
        pltpu.VMEM((_COPIES * _HIST,), jnp.int32),
        pltpu.VMEM((_COPIES * _HIST,), jnp.float32),
        pltpu.VMEM((512,), jnp.int32),
        pltpu.VMEM((512,), jnp.float32),
        pltpu.VMEM((2, 512), jnp.int32),
        pltpu.VMEM((2, 512), jnp.float32),
        pltpu.VMEM((16, 512), jnp.int32),
        pltpu.VMEM((16, 512), jnp.float32),
        pltpu.VMEM((16,), jnp.int32),
        pltpu.VMEM((16,), jnp.float32),
        pltpu.VMEM_SHARED((16, 512), jnp.int32),
        pltpu.VMEM_SHARED((16, 512), jnp.float32),
        pltpu.SemaphoreType.DMA,
        pltpu.SemaphoreType.DMA,
    ]

    zc = jnp.zeros((_NCORE, 512), jnp.int32)
    zs = jnp.zeros((_NCORE, 512), jnp.float32)
    zi = jnp.zeros((_NCORE, 16), jnp.int32)
    zf = jnp.zeros((_NCORE, 16), jnp.float32)

    hc, hs = None, None
    sti, stf = zi, zf
    for l in range(4):
        f = pl.kernel(
            functools.partial(_scan_body, l),
            out_type=hist_out + st_out,
            mesh=mesh,
            compiler_params=params,
            scratch_types=scan_scratch,
        )
        hc, hs, sti_n, stf_n = f(keys,
                                 zc if hc is None else hc,
                                 zs if hs is None else hs,
                                 sti, stf)
        if l > 0:
            sti, stf = sti_n, stf_n

    f = pl.kernel(
        _final_body,
        out_type=jax.ShapeDtypeStruct((16,), jnp.float32),
        mesh=mesh,
        compiler_params=params,
        scratch_types=[
            pltpu.VMEM((512,), jnp.int32),
            pltpu.VMEM((512,), jnp.float32),
            pltpu.VMEM((2, 512), jnp.int32),
            pltpu.VMEM((2, 512), jnp.float32),
            pltpu.VMEM((16,), jnp.int32),
            pltpu.VMEM((16,), jnp.float32),
            pltpu.VMEM((16,), jnp.float32),
        ],
    )
    return f(hc, hs, sti, stf)


def kernel(inputs, targets):
    keys = _compute_keys(inputs, targets)
    out = _sc_select(keys)
    return out[0]


# speedup vs baseline: 1.2202x; 1.0960x over previous
import functools

import jax
import jax.numpy as jnp
from jax import lax
from jax.experimental import pallas as pl
from jax.experimental.pallas import tpu as pltpu
from jax.experimental.pallas import tpu_sc as plsc

N = 2097152
EPS = 1e-07
RATIO = 0.3

_NCORE = 2
_NSUB = 16
_NWORK = _NCORE * _NSUB
_PER_SUB = N // _NWORK
_CHUNK = 8192
_NCHUNK = _PER_SUB // _CHUNK
_VECS = _CHUNK // 16
_HIST = 16 * 512
_UNROLL = 4
_COPIES = 1

_TOPBIT = -(2**31)
_MASK31 = 0x7FFFFFFF


def _loss_key_body(x_ref, z_ref, key_ref):
    x = x_ref[...]
    z = z_ref[...]
    probs = jnp.clip(jax.nn.sigmoid(x), EPS, 1.0 - EPS)
    pt = probs * z + (1.0 - probs) * (1.0 - z)
    bce = jnp.maximum(x, 0.0) - x * z + jnp.log1p(jnp.exp(-jnp.abs(x)))
    pos = z == 1.0
    one_m = 1.0 - pt
    focal = jnp.where(pos, one_m * one_m, one_m)
    alpha = jnp.where(pos, jnp.float32(1.0), jnp.float32(0.5))
    loss = alpha * focal * bce
    bits = lax.bitcast_convert_type(loss, jnp.int32)
    key_ref[...] = jnp.where(pos, bits | _TOPBIT, bits)


def _compute_keys(inputs, targets):
    blk = N // 8
    return pl.pallas_call(
        _loss_key_body,
        grid=(8,),
        in_specs=[
            pl.BlockSpec((blk,), lambda i: (i,)),
            pl.BlockSpec((blk,), lambda i: (i,)),
        ],
        out_specs=pl.BlockSpec((blk,), lambda i: (i,)),
        out_shape=jax.ShapeDtypeStruct((N,), jnp.int32),
    )(inputs, targets)


def _lane_i(lane, v, j):
    return jnp.sum(jnp.where(lane == j, v, jnp.int32(0)))


def _lane_f(lane, v, j):
    return jnp.sum(jnp.where(lane == j, v, jnp.float32(0.0)))


def _revcumsum(v):
    return lax.rev(jnp.cumsum(lax.rev(v, (0,))), (0,))


def _select_level(l, lane, red_c, red_s, st):
    P, c_above, s_above, kk, alive = [list(x) for x in st]
    shift = 24 - 8 * l
    zeros_i = jnp.zeros((16,), jnp.int32)
    zeros_f = jnp.zeros((16,), jnp.float32)
    for cls in range(2):
        base = cls * 256
        chunkV = zeros_i
        chunkS = zeros_f
        for ci in range(16):
            vc = red_c[pl.ds(base + ci * 16, 16)]
            vs = red_s[pl.ds(base + ci * 16, 16)]
            chunkV = jnp.where(lane == ci, jnp.sum(vc), chunkV)
            chunkS = jnp.where(lane == ci, jnp.sum(vs), chunkS)
        if l == 0:
            n_cls = jnp.sum(chunkV)
            alive[cls] = (n_cls > 0).astype(jnp.int32)
            kf = n_cls.astype(jnp.float32) * jnp.float32(RATIO)
            ki = kf.astype(jnp.int32)
            ki = ki - (ki.astype(jnp.float32) > kf).astype(jnp.int32)
            kk[cls] = jnp.maximum(jnp.int32(1), ki)
        r = kk[cls] - c_above[cls]
        SCi = _revcumsum(chunkV)
        SSi = _revcumsum(chunkS)
        I = jnp.maximum(jnp.max(plsc.all_reduce_population_count(SCi >= r)) - 1, 0)
        A_c = _lane_i(lane, SCi - chunkV, I)
        A_s = _lane_f(lane, SSi - chunkS, I)
        c16 = red_c[pl.ds(base + I * 16, 16)]
        s16 = red_s[pl.ds(base + I * 16, 16)]
        W = _revcumsum(c16)
        Ws = _revcumsum(s16)
        jj = jnp.maximum(jnp.max(plsc.all_reduce_population_count((A_c + W) >= r)) - 1, 0)
        B = I * 16 + jj
        cn = c_above[cls] + A_c + _lane_i(lane, W - c16, jj)
        sn = s_above[cls] + A_s + _lane_f(lane, Ws - s16, jj)
        pn = P[cls] | lax.shift_left(B, jnp.int32(shift))
        ok = alive[cls] > 0
        c_above[cls] = jnp.where(ok, cn, c_above[cls])
        s_above[cls] = jnp.where(ok, sn, s_above[cls])
        P[cls] = jnp.where(ok, pn, P[cls])
    return P, c_above, s_above, kk, alive


def _merge_prev(prev_c, prev_s, mp_c, mp_s, red_c, red_s):
    pltpu.sync_copy(prev_c, mp_c)
    pltpu.sync_copy(prev_s, mp_s)

    def _m(j, _):
        red_c[pl.ds(j * 16, 16)] = (mp_c[0, pl.ds(j * 16, 16)]
                                    + mp_c[1, pl.ds(j * 16, 16)])
        red_s[pl.ds(j * 16, 16)] = (mp_s[0, pl.ds(j * 16, 16)]
                                    + mp_s[1, pl.ds(j * 16, 16)])
        return 0

    lax.fori_loop(0, 32, _m, 0)


def _unpack_state(lane, stv_i, stv_f):
    vi = stv_i[...]
    vf = stv_f[...]
    P = [_lane_i(lane, vi, 0), _lane_i(lane, vi, 1)]
    c_above = [_lane_i(lane, vi, 2), _lane_i(lane, vi, 3)]
    kk = [_lane_i(lane, vi, 4), _lane_i(lane, vi, 5)]
    alive = [_lane_i(lane, vi, 6), _lane_i(lane, vi, 7)]
    s_above = [_lane_f(lane, vf, 0), _lane_f(lane, vf, 1)]
    return P, c_above, s_above, kk, alive


def _pack_state(lane, st):
    P, c_above, s_above, kk, alive = st
    vals_i = [P[0], P[1], c_above[0], c_above[1], kk[0], kk[1], alive[0], alive[1]]
    vi = jnp.zeros((16,), jnp.int32)
    for j, v in enumerate(vals_i):
        vi = jnp.where(lane == j, v, vi)
    vf = jnp.zeros((16,), jnp.float32)
    vf = jnp.where(lane == 0, s_above[0], vf)
    vf = jnp.where(lane == 1, s_above[1], vf)
    return vi, vf


def _init_state():
    z = jnp.zeros((), jnp.int32)
    zf = jnp.zeros((), jnp.float32)
    return ([z, z + _TOPBIT], [z, z], [zf, zf], [z, z], [z, z])


def _scan_body(l, keys_hbm, prev_c, prev_s, st_i_in, st_f_in,
               out_c, out_s, st_i_out, st_f_out,
               buf0, buf1, hist_c, hist_s, red_c, red_s, mp_c, mp_s, mc, ms,
               stv_i, stv_f, stage_c, stage_s, sem0, sem1):
    sid = lax.axis_index("s")
    cid = lax.axis_index("c")
    wid = cid * _NSUB + sid
    lane = lax.iota(jnp.int32, 16)
    lane_base = lane * jnp.int32(512)
    ones_i = jnp.ones((16,), jnp.int32)
    zeros_i = jnp.zeros((16,), jnp.int32)
    zeros_f = jnp.zeros((16,), jnp.float32)
    ubase = [lane_base + (u % _COPIES) * _HIST for u in range(_UNROLL)]

    if l == 0:
        st = _init_state()
    else:
        _merge_prev(prev_c, prev_s, mp_c, mp_s, red_c, red_s)
        if l == 1:
            st = _init_state()
        else:
            pltpu.sync_copy(st_i_in.at[0], stv_i)
            pltpu.sync_copy(st_f_in.at[0], stv_f)
            st = _unpack_state(lane, stv_i, stv_f)
        st = _select_level(l - 1, lane, red_c, red_s, st)
        vi, vf = _pack_state(lane, st)
        stv_i[...] = vi
        stv_f[...] = vf

    P = st[0]
    shift = 24 - 8 * l
    mask_hi = _TOPBIT if l == 0 else -(1 << (32 - 8 * l))
    shift_v = jnp.full((16,), shift, jnp.int32)

    def _zero(i, _):
        hist_c[pl.ds(i * 16, 16)] = zeros_i
        hist_s[pl.ds(i * 16, 16)] = zeros_f
        return 0

    lax.fori_loop(0, _COPIES * _HIST // 16, _zero, 0)

    Pn, Pp = P[0], P[1]

    def _scan_buf(buf):
        def _scan(i, _):
            vo = i * (16 * _UNROLL)
            idxs, losses, ms_ = [], [], []
            for u in range(_UNROLL):
                x = buf[pl.ds(vo + u * 16, 16)]
                mn = ((x ^ Pn) & mask_hi) == 0
                mp = ((x ^ Pp) & mask_hi) == 0
                b = lax.shift_right_logical(x, shift_v) & jnp.int32(0xFF)
                idxs.append(ubase[u] + b
                            + jnp.where(mp, jnp.int32(256), jnp.int32(0)))
                ms_.append(mn | mp)
                losses.append(plsc.bitcast(x & _MASK31, jnp.float32))
            for u in range(_UNROLL):
                plsc.addupdate_scatter(hist_c, [idxs[u]], ones_i, mask=ms_[u])
                plsc.addupdate_scatter(hist_s, [idxs[u]], losses[u], mask=ms_[u])
            return 0

        lax.fori_loop(0, _VECS // _UNROLL, _scan, 0)

    def _chunk_slice(c):
        return keys_hbm.at[pl.ds(wid * _PER_SUB + c * _CHUNK, _CHUNK)]

    pltpu.async_copy(_chunk_slice(0), buf0, sem0)

    def _dbl(j, _):
        pltpu.async_copy(_chunk_slice(2 * j + 1), buf1, sem1)
        pltpu.make_async_copy(_chunk_slice(0), buf0, sem0).wait()
        _scan_buf(buf0)
        pltpu.async_copy(_chunk_slice(jnp.minimum(2 * j + 2, _NCHUNK - 1)),
                         buf0, sem0)
        pltpu.make_async_copy(_chunk_slice(0), buf1, sem1).wait()
        _scan_buf(buf1)
        return 0

    lax.fori_loop(0, _NCHUNK // 2, _dbl, 0)
    pltpu.make_async_copy(_chunk_slice(0), buf0, sem0).wait()

    def _fold(i, _):
        o = i * 16
        hist_c[pl.ds(o, 16)] = sum(hist_c[pl.ds(o + u * _HIST, 16)]
                                   for u in range(1, _COPIES)) + hist_c[pl.ds(o, 16)]
        hist_s[pl.ds(o, 16)] = sum(hist_s[pl.ds(o + u * _HIST, 16)]
                                   for u in range(1, _COPIES)) + hist_s[pl.ds(o, 16)]
        return 0

    if _COPIES > 1:
        lax.fori_loop(0, _HIST // 16, _fold, 0)

    def _lred(j, _):
        def _acc(ln, carry):
            ac, asum = carry
            off = ln * jnp.int32(512) + j * 16
            return ac + hist_c[pl.ds(off, 16)], asum + hist_s[pl.ds(off, 16)]

        ac, asum = lax.fori_loop(0, 16, _acc, (zeros_i, zeros_f))
        red_c[pl.ds(j * 16, 16)] = ac
        red_s[pl.ds(j * 16, 16)] = asum
        return 0

    lax.fori_loop(0, 32, _lred, 0)

    pltpu.sync_copy(red_c, stage_c.at[sid])
    pltpu.sync_copy(red_s, stage_s.at[sid])
    plsc.subcore_barrier()

    @pl.when(sid == 0)
    def _():
        def _gagg(j, _):
            def _acc(s, carry):
                ac, asum = carry
                return (ac + mc[s, pl.ds(j * 16, 16)],
                        asum + ms[s, pl.ds(j * 16, 16)])

            ac, asum = lax.fori_loop(0, 16, _acc, (zeros_i, zeros_f))
            red_c[pl.ds(j * 16, 16)] = ac
            red_s[pl.ds(j * 16, 16)] = asum
            return 0

        pltpu.sync_copy(stage_c, mc)
        pltpu.sync_copy(stage_s, ms)
        lax.fori_loop(0, 32, _gagg, 0)
        pltpu.sync_copy(red_c, out_c.at[cid])
        pltpu.sync_copy(red_s, out_s.at[cid])
        if l > 0:
            pltpu.sync_copy(stv_i, st_i_out.at[cid])
            pltpu.sync_copy(stv_f, st_f_out.at[cid])


def _final_body(prev_c, prev_s, st_i_in, st_f_in, out_hbm,
                red_c, red_s, mp_c, mp_s, stv_i, stv_f, outv):
    sid = lax.axis_index("s")
    cid = lax.axis_index("c")
    lane = lax.iota(jnp.int32, 16)

    @pl.when((sid == 0) & (cid == 0))
    def _():
        _merge_prev(prev_c, prev_s, mp_c, mp_s, red_c, red_s)
        pltpu.sync_copy(st_i_in.at[0], stv_i)
        pltpu.sync_copy(st_f_in.at[0], stv_f)
        st = _unpack_state(lane, stv_i, stv_f)
        P, c_above, s_above, kk, alive = _select_level(3, lane, red_c, red_s, st)
        num = jnp.zeros((16,), jnp.float32)
        den = jnp.zeros((), jnp.float32)
        for cls in range(2):
            t_bits = jnp.zeros((16,), jnp.int32) + (P[cls] & _MASK31)
            t_f = plsc.bitcast(t_bits, jnp.float32)
            contrib = s_above[cls] + (kk[cls] - c_above[cls]).astype(jnp.float32) * t_f
            af = alive[cls].astype(jnp.float32)
            num = num + af * contrib
            den = den + af * kk[cls].astype(jnp.float32)
        outv[...] = num / den
        pltpu.sync_copy(outv, out_hbm)


def _sc_select(keys):
    mesh = plsc.VectorSubcoreMesh(core_axis_name="c", subcore_axis_name="s",
                                  num_cores=_NCORE)
    params = pltpu.CompilerParams(needs_layout_passes=False)
    hist_out = (jax.ShapeDtypeStruct((_NCORE, 512), jnp.int32),
                jax.ShapeDtypeStruct((_NCORE, 512), jnp.float32))
    st_out = (jax.ShapeDtypeStruct((_NCORE, 16), jnp.int32),
              jax.ShapeDtypeStruct((_NCORE, 16), jnp.float32))
    scan_scratch = [
        pltpu.VMEM((_CHUNK,), jnp.int32),
        pltpu.VMEM((_CHUNK,), jnp.int32),
        pltpu.VMEM((_COPIES * _HIST,), jnp.int32),
        pltpu.VMEM((_COPIES * _HIST,), jnp.float32),
        pltpu.VMEM((512,), jnp.int32),
        pltpu.VMEM((512,), jnp.float32),
        pltpu.VMEM((2, 512), jnp.int32),
        pltpu.VMEM((2, 512), jnp.float32),
        pltpu.VMEM((16, 512), jnp.int32),
        pltpu.VMEM((16, 512), jnp.float32),
        pltpu.VMEM((16,), jnp.int32),
        pltpu.VMEM((16,), jnp.float32),
        pltpu.VMEM_SHARED((16, 512), jnp.int32),
        pltpu.VMEM_SHARED((16, 512), jnp.float32),
        pltpu.SemaphoreType.DMA,
        pltpu.SemaphoreType.DMA,
    ]

    zc = jnp.zeros((_NCORE, 512), jnp.int32)
    zs = jnp.zeros((_NCORE, 512), jnp.float32)
    zi = jnp.zeros((_NCORE, 16), jnp.int32)
    zf = jnp.zeros((_NCORE, 16), jnp.float32)

    hc, hs = None, None
    sti, stf = zi, zf
    for l in range(4):
        f = pl.kernel(
            functools.partial(_scan_body, l),
            out_type=hist_out + st_out,
            mesh=mesh,
            compiler_params=params,
            scratch_types=scan_scratch,
        )
        hc, hs, sti_n, stf_n = f(keys,
                                 zc if hc is None else hc,
                                 zs if hs is None else hs,
                                 sti, stf)
        if l > 0:
            sti, stf = sti_n, stf_n

    f = pl.kernel(
        _final_body,
        out_type=jax.ShapeDtypeStruct((16,), jnp.float32),
        mesh=mesh,
        compiler_params=params,
        scratch_types=[
            pltpu.VMEM((512,), jnp.int32),
            pltpu.VMEM((512,), jnp.float32),
            pltpu.VMEM((2, 512), jnp.int32),
            pltpu.VMEM((2, 512), jnp.float32),
            pltpu.VMEM((16,), jnp.int32),
            pltpu.VMEM((16,), jnp.float32),
            pltpu.VMEM((16,), jnp.float32),
        ],
    )
    return f(hc, hs, sti, stf)


def kernel(inputs, targets):
    keys = _compute_keys(inputs, targets)
    out = _sc_select(keys)
    return out[0]


# speedup vs baseline: 1.3095x; 1.0732x over previous
import functools

import jax
import jax.numpy as jnp
from jax import lax
from jax.experimental import pallas as pl
from jax.experimental.pallas import tpu as pltpu
from jax.experimental.pallas import tpu_sc as plsc

N = 2097152
EPS = 1e-07
RATIO = 0.3

_NCORE = 2
_NSUB = 16
_NWORK = _NCORE * _NSUB
_PER_SUB = N // _NWORK
_CHUNK = 8192
_NCHUNK = _PER_SUB // _CHUNK
_VECS = _CHUNK // 16
_HIST = 16 * 512
_UNROLL = 8
_COPIES = 1

_TOPBIT = -(2**31)
_MASK31 = 0x7FFFFFFF


def _loss_key_body(x_ref, z_ref, key_ref):
    x = x_ref[...]
    z = z_ref[...]
    probs = jnp.clip(jax.nn.sigmoid(x), EPS, 1.0 - EPS)
    pt = probs * z + (1.0 - probs) * (1.0 - z)
    bce = jnp.maximum(x, 0.0) - x * z + jnp.log1p(jnp.exp(-jnp.abs(x)))
    pos = z == 1.0
    one_m = 1.0 - pt
    focal = jnp.where(pos, one_m * one_m, one_m)
    alpha = jnp.where(pos, jnp.float32(1.0), jnp.float32(0.5))
    loss = alpha * focal * bce
    bits = lax.bitcast_convert_type(loss, jnp.int32)
    key_ref[...] = jnp.where(pos, bits | _TOPBIT, bits)


def _compute_keys(inputs, targets):
    blk = N // 8
    return pl.pallas_call(
        _loss_key_body,
        grid=(8,),
        in_specs=[
            pl.BlockSpec((blk,), lambda i: (i,)),
            pl.BlockSpec((blk,), lambda i: (i,)),
        ],
        out_specs=pl.BlockSpec((blk,), lambda i: (i,)),
        out_shape=jax.ShapeDtypeStruct((N,), jnp.int32),
    )(inputs, targets)


def _lane_i(lane, v, j):
    return jnp.sum(jnp.where(lane == j, v, jnp.int32(0)))


def _lane_f(lane, v, j):
    return jnp.sum(jnp.where(lane == j, v, jnp.float32(0.0)))


def _revcumsum(v):
    return lax.rev(jnp.cumsum(lax.rev(v, (0,))), (0,))


def _select_level(l, lane, red_c, red_s, st):
    P, c_above, s_above, kk, alive = [list(x) for x in st]
    shift = 24 - 8 * l
    zeros_i = jnp.zeros((16,), jnp.int32)
    zeros_f = jnp.zeros((16,), jnp.float32)
    for cls in range(2):
        base = cls * 256
        chunkV = zeros_i
        chunkS = zeros_f
        for ci in range(16):
            vc = red_c[pl.ds(base + ci * 16, 16)]
            vs = red_s[pl.ds(base + ci * 16, 16)]
            chunkV = jnp.where(lane == ci, jnp.sum(vc), chunkV)
            chunkS = jnp.where(lane == ci, jnp.sum(vs), chunkS)
        if l == 0:
            n_cls = jnp.sum(chunkV)
            alive[cls] = (n_cls > 0).astype(jnp.int32)
            kf = n_cls.astype(jnp.float32) * jnp.float32(RATIO)
            ki = kf.astype(jnp.int32)
            ki = ki - (ki.astype(jnp.float32) > kf).astype(jnp.int32)
            kk[cls] = jnp.maximum(jnp.int32(1), ki)
        r = kk[cls] - c_above[cls]
        SCi = _revcumsum(chunkV)
        SSi = _revcumsum(chunkS)
        I = jnp.maximum(jnp.max(plsc.all_reduce_population_count(SCi >= r)) - 1, 0)
        A_c = _lane_i(lane, SCi - chunkV, I)
        A_s = _lane_f(lane, SSi - chunkS, I)
        c16 = red_c[pl.ds(base + I * 16, 16)]
        s16 = red_s[pl.ds(base + I * 16, 16)]
        W = _revcumsum(c16)
        Ws = _revcumsum(s16)
        jj = jnp.maximum(jnp.max(plsc.all_reduce_population_count((A_c + W) >= r)) - 1, 0)
        B = I * 16 + jj
        cn = c_above[cls] + A_c + _lane_i(lane, W - c16, jj)
        sn = s_above[cls] + A_s + _lane_f(lane, Ws - s16, jj)
        pn = P[cls] | lax.shift_left(B, jnp.int32(shift))
        ok = alive[cls] > 0
        c_above[cls] = jnp.where(ok, cn, c_above[cls])
        s_above[cls] = jnp.where(ok, sn, s_above[cls])
        P[cls] = jnp.where(ok, pn, P[cls])
    return P, c_above, s_above, kk, alive


def _merge_prev(prev_c, prev_s, mp_c, mp_s, red_c, red_s):
    pltpu.sync_copy(prev_c, mp_c)
    pltpu.sync_copy(prev_s, mp_s)

    def _m(j, _):
        red_c[pl.ds(j * 16, 16)] = (mp_c[0, pl.ds(j * 16, 16)]
                                    + mp_c[1, pl.ds(j * 16, 16)])
        red_s[pl.ds(j * 16, 16)] = (mp_s[0, pl.ds(j * 16, 16)]
                                    + mp_s[1, pl.ds(j * 16, 16)])
        return 0

    lax.fori_loop(0, 32, _m, 0)


def _unpack_state(lane, stv_i, stv_f):
    vi = stv_i[...]
    vf = stv_f[...]
    P = [_lane_i(lane, vi, 0), _lane_i(lane, vi, 1)]
    c_above = [_lane_i(lane, vi, 2), _lane_i(lane, vi, 3)]
    kk = [_lane_i(lane, vi, 4), _lane_i(lane, vi, 5)]
    alive = [_lane_i(lane, vi, 6), _lane_i(lane, vi, 7)]
    s_above = [_lane_f(lane, vf, 0), _lane_f(lane, vf, 1)]
    return P, c_above, s_above, kk, alive


def _pack_state(lane, st):
    P, c_above, s_above, kk, alive = st
    vals_i = [P[0], P[1], c_above[0], c_above[1], kk[0], kk[1], alive[0], alive[1]]
    vi = jnp.zeros((16,), jnp.int32)
    for j, v in enumerate(vals_i):
        vi = jnp.where(lane == j, v, vi)
    vf = jnp.zeros((16,), jnp.float32)
    vf = jnp.where(lane == 0, s_above[0], vf)
    vf = jnp.where(lane == 1, s_above[1], vf)
    return vi, vf


def _init_state():
    z = jnp.zeros((), jnp.int32)
    zf = jnp.zeros((), jnp.float32)
    return ([z, z + _TOPBIT], [z, z], [zf, zf], [z, z], [z, z])


def _scan_body(l, keys_hbm, prev_c, prev_s, st_i_in, st_f_in,
               out_c, out_s, st_i_out, st_f_out,
               buf0, buf1, hist_c, hist_s, red_c, red_s, mp_c, mp_s, mc, ms,
               stv_i, stv_f, stage_c, stage_s, sem0, sem1):
    sid = lax.axis_index("s")
    cid = lax.axis_index("c")
    wid = cid * _NSUB + sid
    lane = lax.iota(jnp.int32, 16)
    lane_base = lane * jnp.int32(512)
    ones_i = jnp.ones((16,), jnp.int32)
    zeros_i = jnp.zeros((16,), jnp.int32)
    zeros_f = jnp.zeros((16,), jnp.float32)
    ubase = [lane_base + (u % _COPIES) * _HIST for u in range(_UNROLL)]

    if l == 0:
        st = _init_state()
    else:
        _merge_prev(prev_c, prev_s, mp_c, mp_s, red_c, red_s)
        if l == 1:
            st = _init_state()
        else:
            pltpu.sync_copy(st_i_in.at[0], stv_i)
            pltpu.sync_copy(st_f_in.at[0], stv_f)
            st = _unpack_state(lane, stv_i, stv_f)
        st = _select_level(l - 1, lane, red_c, red_s, st)
        vi, vf = _pack_state(lane, st)
        stv_i[...] = vi
        stv_f[...] = vf

    P = st[0]
    shift = 24 - 8 * l
    mask_hi = _TOPBIT if l == 0 else -(1 << (32 - 8 * l))
    shift_v = jnp.full((16,), shift, jnp.int32)

    def _zero(i, _):
        hist_c[pl.ds(i * 16, 16)] = zeros_i
        hist_s[pl.ds(i * 16, 16)] = zeros_f
        return 0

    lax.fori_loop(0, _COPIES * _HIST // 16, _zero, 0)

    Pn, Pp = P[0], P[1]

    def _scan_buf(buf):
        def _scan(i, _):
            vo = i * (16 * _UNROLL)
            idxs, losses, ms_ = [], [], []
            for u in range(_UNROLL):
                x = buf[pl.ds(vo + u * 16, 16)]
                mn = ((x ^ Pn) & mask_hi) == 0
                mp = ((x ^ Pp) & mask_hi) == 0
                b = lax.shift_right_logical(x, shift_v) & jnp.int32(0xFF)
                idxs.append(ubase[u] + b
                            + jnp.where(mp, jnp.int32(256), jnp.int32(0)))
                ms_.append(mn | mp)
                losses.append(plsc.bitcast(x & _MASK31, jnp.float32))
            for u in range(_UNROLL):
                plsc.addupdate_scatter(hist_c, [idxs[u]], ones_i, mask=ms_[u])
                plsc.addupdate_scatter(hist_s, [idxs[u]], losses[u], mask=ms_[u])
            return 0

        lax.fori_loop(0, _VECS // _UNROLL, _scan, 0)

    def _chunk_slice(c):
        return keys_hbm.at[pl.ds(wid * _PER_SUB + c * _CHUNK, _CHUNK)]

    pltpu.async_copy(_chunk_slice(0), buf0, sem0)

    def _dbl(j, _):
        pltpu.async_copy(_chunk_slice(2 * j + 1), buf1, sem1)
        pltpu.make_async_copy(_chunk_slice(0), buf0, sem0).wait()
        _scan_buf(buf0)
        pltpu.async_copy(_chunk_slice(jnp.minimum(2 * j + 2, _NCHUNK - 1)),
                         buf0, sem0)
        pltpu.make_async_copy(_chunk_slice(0), buf1, sem1).wait()
        _scan_buf(buf1)
        return 0

    lax.fori_loop(0, _NCHUNK // 2, _dbl, 0)
    pltpu.make_async_copy(_chunk_slice(0), buf0, sem0).wait()

    def _fold(i, _):
        o = i * 16
        hist_c[pl.ds(o, 16)] = sum(hist_c[pl.ds(o + u * _HIST, 16)]
                                   for u in range(1, _COPIES)) + hist_c[pl.ds(o, 16)]
        hist_s[pl.ds(o, 16)] = sum(hist_s[pl.ds(o + u * _HIST, 16)]
                                   for u in range(1, _COPIES)) + hist_s[pl.ds(o, 16)]
        return 0

    if _COPIES > 1:
        lax.fori_loop(0, _HIST // 16, _fold, 0)

    def _lred(j, _):
        def _acc(ln, carry):
            ac, asum = carry
            off = ln * jnp.int32(512) + j * 16
            return ac + hist_c[pl.ds(off, 16)], asum + hist_s[pl.ds(off, 16)]

        ac, asum = lax.fori_loop(0, 16, _acc, (zeros_i, zeros_f))
        red_c[pl.ds(j * 16, 16)] = ac
        red_s[pl.ds(j * 16, 16)] = asum
        return 0

    lax.fori_loop(0, 32, _lred, 0)

    pltpu.sync_copy(red_c, stage_c.at[sid])
    pltpu.sync_copy(red_s, stage_s.at[sid])
    plsc.subcore_barrier()

    @pl.when(sid == 0)
    def _():
        def _gagg(j, _):
            def _acc(s, carry):
                ac, asum = carry
                return (ac + mc[s, pl.ds(j * 16, 16)],
                        asum + ms[s, pl.ds(j * 16, 16)])

            ac, asum = lax.fori_loop(0, 16, _acc, (zeros_i, zeros_f))
            red_c[pl.ds(j * 16, 16)] = ac
            red_s[pl.ds(j * 16, 16)] = asum
            return 0

        pltpu.sync_copy(stage_c, mc)
        pltpu.sync_copy(stage_s, ms)
        lax.fori_loop(0, 32, _gagg, 0)
        pltpu.sync_copy(red_c, out_c.at[cid])
        pltpu.sync_copy(red_s, out_s.at[cid])
        if l > 0:
            pltpu.sync_copy(stv_i, st_i_out.at[cid])
            pltpu.sync_copy(stv_f, st_f_out.at[cid])


def _final_body(prev_c, prev_s, st_i_in, st_f_in, out_hbm,
                red_c, red_s, mp_c, mp_s, stv_i, stv_f, outv):
    sid = lax.axis_index("s")
    cid = lax.axis_index("c")
    lane = lax.iota(jnp.int32, 16)

    @pl.when((sid == 0) & (cid == 0))
    def _():
        _merge_prev(prev_c, prev_s, mp_c, mp_s, red_c, red_s)
        pltpu.sync_copy(st_i_in.at[0], stv_i)
        pltpu.sync_copy(st_f_in.at[0], stv_f)
        st = _unpack_state(lane, stv_i, stv_f)
        P, c_above, s_above, kk, alive = _select_level(3, lane, red_c, red_s, st)
        num = jnp.zeros((16,), jnp.float32)
        den = jnp.zeros((), jnp.float32)
        for cls in range(2):
            t_bits = jnp.zeros((16,), jnp.int32) + (P[cls] & _MASK31)
            t_f = plsc.bitcast(t_bits, jnp.float32)
            contrib = s_above[cls] + (kk[cls] - c_above[cls]).astype(jnp.float32) * t_f
            af = alive[cls].astype(jnp.float32)
            num = num + af * contrib
            den = den + af * kk[cls].astype(jnp.float32)
        outv[...] = num / den
        pltpu.sync_copy(outv, out_hbm)


def _sc_select(keys):
    mesh = plsc.VectorSubcoreMesh(core_axis_name="c", subcore_axis_name="s",
                                  num_cores=_NCORE)
    params = pltpu.CompilerParams(needs_layout_passes=False)
    hist_out = (jax.ShapeDtypeStruct((_NCORE, 512), jnp.int32),
                jax.ShapeDtypeStruct((_NCORE, 512), jnp.float32))
    st_out = (jax.ShapeDtypeStruct((_NCORE, 16), jnp.int32),
              jax.ShapeDtypeStruct((_NCORE, 16), jnp.float32))
    scan_scratch = [
        pltpu.VMEM((_CHUNK,), jnp.int32),
        pltpu.VMEM((_CHUNK,), jnp.int32),
        pltpu.VMEM((_COPIES * _HIST,), jnp.int32),
        pltpu.VMEM((_COPIES * _HIST,), jnp.float32),
        pltpu.VMEM((512,), jnp.int32),
        pltpu.VMEM((512,), jnp.float32),
        pltpu.VMEM((2, 512), jnp.int32),
        pltpu.VMEM((2, 512), jnp.float32),
        pltpu.VMEM((16, 512), jnp.int32),
        pltpu.VMEM((16, 512), jnp.float32),
        pltpu.VMEM((16,), jnp.int32),
        pltpu.VMEM((16,), jnp.float32),
        pltpu.VMEM_SHARED((16, 512), jnp.int32),
        pltpu.VMEM_SHARED((16, 512), jnp.float32),
        pltpu.SemaphoreType.DMA,
        pltpu.SemaphoreType.DMA,
    ]

    zc = jnp.zeros((_NCORE, 512), jnp.int32)
    zs = jnp.zeros((_NCORE, 512), jnp.float32)
    zi = jnp.zeros((_NCORE, 16), jnp.int32)
    zf = jnp.zeros((_NCORE, 16), jnp.float32)

    hc, hs = None, None
    sti, stf = zi, zf
    for l in range(4):
        f = pl.kernel(
            functools.partial(_scan_body, l),
            out_type=hist_out + st_out,
            mesh=mesh,
            compiler_params=params,
            scratch_types=scan_scratch,
        )
        hc, hs, sti_n, stf_n = f(keys,
                                 zc if hc is None else hc,
                                 zs if hs is None else hs,
                                 sti, stf)
        if l > 0:
            sti, stf = sti_n, stf_n

    f = pl.kernel(
        _final_body,
        out_type=jax.ShapeDtypeStruct((16,), jnp.float32),
        mesh=mesh,
        compiler_params=params,
        scratch_types=[
            pltpu.VMEM((512,), jnp.int32),
            pltpu.VMEM((512,), jnp.float32),
            pltpu.VMEM((2, 512), jnp.int32),
            pltpu.VMEM((2, 512), jnp.float32),
            pltpu.VMEM((16,), jnp.int32),
            pltpu.VMEM((16,), jnp.float32),
            pltpu.VMEM((16,), jnp.float32),
        ],
    )
    return f(hc, hs, sti, stf)


def kernel(inputs, targets):
    keys = _compute_keys(inputs, targets)
    out = _sc_select(keys)
    return out[0]


# speedup vs baseline: 1.3431x; 1.0257x over previous
import functools

import jax
import jax.numpy as jnp
from jax import lax
from jax.experimental import pallas as pl
from jax.experimental.pallas import tpu as pltpu
from jax.experimental.pallas import tpu_sc as plsc

N = 2097152
EPS = 1e-07
RATIO = 0.3

_NCORE = 2
_NSUB = 16
_NWORK = _NCORE * _NSUB
_PER_SUB = N // _NWORK
_CHUNK = 8192
_NCHUNK = _PER_SUB // _CHUNK
_VECS = _CHUNK // 16
_HIST = 16 * 512
_UNROLL = 16
_COPIES = 1

_TOPBIT = -(2**31)
_MASK31 = 0x7FFFFFFF


def _loss_key_body(x_ref, z_ref, key_ref):
    x = x_ref[...]
    z = z_ref[...]
    probs = jnp.clip(jax.nn.sigmoid(x), EPS, 1.0 - EPS)
    pt = probs * z + (1.0 - probs) * (1.0 - z)
    bce = jnp.maximum(x, 0.0) - x * z + jnp.log1p(jnp.exp(-jnp.abs(x)))
    pos = z == 1.0
    one_m = 1.0 - pt
    focal = jnp.where(pos, one_m * one_m, one_m)
    alpha = jnp.where(pos, jnp.float32(1.0), jnp.float32(0.5))
    loss = alpha * focal * bce
    bits = lax.bitcast_convert_type(loss, jnp.int32)
    key_ref[...] = jnp.where(pos, bits | _TOPBIT, bits)


def _compute_keys(inputs, targets):
    blk = N // 8
    return pl.pallas_call(
        _loss_key_body,
        grid=(8,),
        in_specs=[
            pl.BlockSpec((blk,), lambda i: (i,)),
            pl.BlockSpec((blk,), lambda i: (i,)),
        ],
        out_specs=pl.BlockSpec((blk,), lambda i: (i,)),
        out_shape=jax.ShapeDtypeStruct((N,), jnp.int32),
    )(inputs, targets)


def _lane_i(lane, v, j):
    return jnp.sum(jnp.where(lane == j, v, jnp.int32(0)))


def _lane_f(lane, v, j):
    return jnp.sum(jnp.where(lane == j, v, jnp.float32(0.0)))


def _revcumsum(v):
    return lax.rev(jnp.cumsum(lax.rev(v, (0,))), (0,))


def _select_level(l, lane, red_c, red_s, st):
    P, c_above, s_above, kk, alive = [list(x) for x in st]
    shift = 24 - 8 * l
    zeros_i = jnp.zeros((16,), jnp.int32)
    zeros_f = jnp.zeros((16,), jnp.float32)
    for cls in range(2):
        base = cls * 256
        chunkV = zeros_i
        chunkS = zeros_f
        for ci in range(16):
            vc = red_c[pl.ds(base + ci * 16, 16)]
            vs = red_s[pl.ds(base + ci * 16, 16)]
            chunkV = jnp.where(lane == ci, jnp.sum(vc), chunkV)
            chunkS = jnp.where(lane == ci, jnp.sum(vs), chunkS)
        if l == 0:
            n_cls = jnp.sum(chunkV)
            alive[cls] = (n_cls > 0).astype(jnp.int32)
            kf = n_cls.astype(jnp.float32) * jnp.float32(RATIO)
            ki = kf.astype(jnp.int32)
            ki = ki - (ki.astype(jnp.float32) > kf).astype(jnp.int32)
            kk[cls] = jnp.maximum(jnp.int32(1), ki)
        r = kk[cls] - c_above[cls]
        SCi = _revcumsum(chunkV)
        SSi = _revcumsum(chunkS)
        I = jnp.maximum(jnp.max(plsc.all_reduce_population_count(SCi >= r)) - 1, 0)
        A_c = _lane_i(lane, SCi - chunkV, I)
        A_s = _lane_f(lane, SSi - chunkS, I)
        c16 = red_c[pl.ds(base + I * 16, 16)]
        s16 = red_s[pl.ds(base + I * 16, 16)]
        W = _revcumsum(c16)
        Ws = _revcumsum(s16)
        jj = jnp.maximum(jnp.max(plsc.all_reduce_population_count((A_c + W) >= r)) - 1, 0)
        B = I * 16 + jj
        cn = c_above[cls] + A_c + _lane_i(lane, W - c16, jj)
        sn = s_above[cls] + A_s + _lane_f(lane, Ws - s16, jj)
        pn = P[cls] | lax.shift_left(B, jnp.int32(shift))
        ok = alive[cls] > 0
        c_above[cls] = jnp.where(ok, cn, c_above[cls])
        s_above[cls] = jnp.where(ok, sn, s_above[cls])
        P[cls] = jnp.where(ok, pn, P[cls])
    return P, c_above, s_above, kk, alive


def _merge_prev(prev_c, prev_s, mp_c, mp_s, red_c, red_s):
    pltpu.sync_copy(prev_c, mp_c)
    pltpu.sync_copy(prev_s, mp_s)

    def _m(j, _):
        red_c[pl.ds(j * 16, 16)] = (mp_c[0, pl.ds(j * 16, 16)]
                                    + mp_c[1, pl.ds(j * 16, 16)])
        red_s[pl.ds(j * 16, 16)] = (mp_s[0, pl.ds(j * 16, 16)]
                                    + mp_s[1, pl.ds(j * 16, 16)])
        return 0

    lax.fori_loop(0, 32, _m, 0)


def _unpack_state(lane, stv_i, stv_f):
    vi = stv_i[...]
    vf = stv_f[...]
    P = [_lane_i(lane, vi, 0), _lane_i(lane, vi, 1)]
    c_above = [_lane_i(lane, vi, 2), _lane_i(lane, vi, 3)]
    kk = [_lane_i(lane, vi, 4), _lane_i(lane, vi, 5)]
    alive = [_lane_i(lane, vi, 6), _lane_i(lane, vi, 7)]
    s_above = [_lane_f(lane, vf, 0), _lane_f(lane, vf, 1)]
    return P, c_above, s_above, kk, alive


def _pack_state(lane, st):
    P, c_above, s_above, kk, alive = st
    vals_i = [P[0], P[1], c_above[0], c_above[1], kk[0], kk[1], alive[0], alive[1]]
    vi = jnp.zeros((16,), jnp.int32)
    for j, v in enumerate(vals_i):
        vi = jnp.where(lane == j, v, vi)
    vf = jnp.zeros((16,), jnp.float32)
    vf = jnp.where(lane == 0, s_above[0], vf)
    vf = jnp.where(lane == 1, s_above[1], vf)
    return vi, vf


def _init_state():
    z = jnp.zeros((), jnp.int32)
    zf = jnp.zeros((), jnp.float32)
    return ([z, z + _TOPBIT], [z, z], [zf, zf], [z, z], [z, z])


def _scan_body(l, keys_hbm, prev_c, prev_s, st_i_in, st_f_in,
               out_c, out_s, st_i_out, st_f_out,
               buf0, buf1, hist_c, hist_s, red_c, red_s, mp_c, mp_s, mc, ms,
               stv_i, stv_f, stage_c, stage_s, sem0, sem1):
    sid = lax.axis_index("s")
    cid = lax.axis_index("c")
    wid = cid * _NSUB + sid
    lane = lax.iota(jnp.int32, 16)
    lane_base = lane * jnp.int32(512)
    ones_i = jnp.ones((16,), jnp.int32)
    zeros_i = jnp.zeros((16,), jnp.int32)
    zeros_f = jnp.zeros((16,), jnp.float32)
    ubase = [lane_base + (u % _COPIES) * _HIST for u in range(_UNROLL)]

    if l == 0:
        st = _init_state()
    else:
        _merge_prev(prev_c, prev_s, mp_c, mp_s, red_c, red_s)
        if l == 1:
            st = _init_state()
        else:
            pltpu.sync_copy(st_i_in.at[0], stv_i)
            pltpu.sync_copy(st_f_in.at[0], stv_f)
            st = _unpack_state(lane, stv_i, stv_f)
        st = _select_level(l - 1, lane, red_c, red_s, st)
        vi, vf = _pack_state(lane, st)
        stv_i[...] = vi
        stv_f[...] = vf

    P = st[0]
    shift = 24 - 8 * l
    mask_hi = _TOPBIT if l == 0 else -(1 << (32 - 8 * l))
    shift_v = jnp.full((16,), shift, jnp.int32)

    def _zero(i, _):
        hist_c[pl.ds(i * 16, 16)] = zeros_i
        hist_s[pl.ds(i * 16, 16)] = zeros_f
        return 0

    lax.fori_loop(0, _COPIES * _HIST // 16, _zero, 0)

    Pn, Pp = P[0], P[1]

    def _scan_buf(buf):
        def _scan(i, _):
            vo = i * (16 * _UNROLL)
            idxs, losses, ms_ = [], [], []
            for u in range(_UNROLL):
                x = buf[pl.ds(vo + u * 16, 16)]
                mn = ((x ^ Pn) & mask_hi) == 0
                mp = ((x ^ Pp) & mask_hi) == 0
                b = lax.shift_right_logical(x, shift_v) & jnp.int32(0xFF)
                idxs.append(ubase[u] + b
                            + jnp.where(mp, jnp.int32(256), jnp.int32(0)))
                ms_.append(mn | mp)
                losses.append(plsc.bitcast(x & _MASK31, jnp.float32))
            for u in range(_UNROLL):
                plsc.addupdate_scatter(hist_c, [idxs[u]], ones_i, mask=ms_[u])
                plsc.addupdate_scatter(hist_s, [idxs[u]], losses[u], mask=ms_[u])
            return 0

        lax.fori_loop(0, _VECS // _UNROLL, _scan, 0)

    def _chunk_slice(c):
        return keys_hbm.at[pl.ds(wid * _PER_SUB + c * _CHUNK, _CHUNK)]

    pltpu.async_copy(_chunk_slice(0), buf0, sem0)

    def _dbl(j, _):
        pltpu.async_copy(_chunk_slice(2 * j + 1), buf1, sem1)
        pltpu.make_async_copy(_chunk_slice(0), buf0, sem0).wait()
        _scan_buf(buf0)
        pltpu.async_copy(_chunk_slice(jnp.minimum(2 * j + 2, _NCHUNK - 1)),
                         buf0, sem0)
        pltpu.make_async_copy(_chunk_slice(0), buf1, sem1).wait()
        _scan_buf(buf1)
        return 0

    lax.fori_loop(0, _NCHUNK // 2, _dbl, 0)
    pltpu.make_async_copy(_chunk_slice(0), buf0, sem0).wait()

    def _fold(i, _):
        o = i * 16
        hist_c[pl.ds(o, 16)] = sum(hist_c[pl.ds(o + u * _HIST, 16)]
                                   for u in range(1, _COPIES)) + hist_c[pl.ds(o, 16)]
        hist_s[pl.ds(o, 16)] = sum(hist_s[pl.ds(o + u * _HIST, 16)]
                                   for u in range(1, _COPIES)) + hist_s[pl.ds(o, 16)]
        return 0

    if _COPIES > 1:
        lax.fori_loop(0, _HIST // 16, _fold, 0)

    def _lred(j, _):
        def _acc(ln, carry):
            ac, asum = carry
            off = ln * jnp.int32(512) + j * 16
            return ac + hist_c[pl.ds(off, 16)], asum + hist_s[pl.ds(off, 16)]

        ac, asum = lax.fori_loop(0, 16, _acc, (zeros_i, zeros_f))
        red_c[pl.ds(j * 16, 16)] = ac
        red_s[pl.ds(j * 16, 16)] = asum
        return 0

    lax.fori_loop(0, 32, _lred, 0)

    pltpu.sync_copy(red_c, stage_c.at[sid])
    pltpu.sync_copy(red_s, stage_s.at[sid])
    plsc.subcore_barrier()

    @pl.when(sid == 0)
    def _():
        def _gagg(j, _):
            def _acc(s, carry):
                ac, asum = carry
                return (ac + mc[s, pl.ds(j * 16, 16)],
                        asum + ms[s, pl.ds(j * 16, 16)])

            ac, asum = lax.fori_loop(0, 16, _acc, (zeros_i, zeros_f))
            red_c[pl.ds(j * 16, 16)] = ac
            red_s[pl.ds(j * 16, 16)] = asum
            return 0

        pltpu.sync_copy(stage_c, mc)
        pltpu.sync_copy(stage_s, ms)
        lax.fori_loop(0, 32, _gagg, 0)
        pltpu.sync_copy(red_c, out_c.at[cid])
        pltpu.sync_copy(red_s, out_s.at[cid])
        if l > 0:
            pltpu.sync_copy(stv_i, st_i_out.at[cid])
            pltpu.sync_copy(stv_f, st_f_out.at[cid])


def _final_body(prev_c, prev_s, st_i_in, st_f_in, out_hbm,
                red_c, red_s, mp_c, mp_s, stv_i, stv_f, outv):
    sid = lax.axis_index("s")
    cid = lax.axis_index("c")
    lane = lax.iota(jnp.int32, 16)

    @pl.when((sid == 0) & (cid == 0))
    def _():
        _merge_prev(prev_c, prev_s, mp_c, mp_s, red_c, red_s)
        pltpu.sync_copy(st_i_in.at[0], stv_i)
        pltpu.sync_copy(st_f_in.at[0], stv_f)
        st = _unpack_state(lane, stv_i, stv_f)
        P, c_above, s_above, kk, alive = _select_level(3, lane, red_c, red_s, st)
        num = jnp.zeros((16,), jnp.float32)
        den = jnp.zeros((), jnp.float32)
        for cls in range(2):
            t_bits = jnp.zeros((16,), jnp.int32) + (P[cls] & _MASK31)
            t_f = plsc.bitcast(t_bits, jnp.float32)
            contrib = s_above[cls] + (kk[cls] - c_above[cls]).astype(jnp.float32) * t_f
            af = alive[cls].astype(jnp.float32)
            num = num + af * contrib
            den = den + af * kk[cls].astype(jnp.float32)
        outv[...] = num / den
        pltpu.sync_copy(outv, out_hbm)


def _sc_select(keys):
    mesh = plsc.VectorSubcoreMesh(core_axis_name="c", subcore_axis_name="s",
                                  num_cores=_NCORE)
    params = pltpu.CompilerParams(needs_layout_passes=False)
    hist_out = (jax.ShapeDtypeStruct((_NCORE, 512), jnp.int32),
                jax.ShapeDtypeStruct((_NCORE, 512), jnp.float32))
    st_out = (jax.ShapeDtypeStruct((_NCORE, 16), jnp.int32),
              jax.ShapeDtypeStruct((_NCORE, 16), jnp.float32))
    scan_scratch = [
        pltpu.VMEM((_CHUNK,), jnp.int32),
        pltpu.VMEM((_CHUNK,), jnp.int32),
        pltpu.VMEM((_COPIES * _HIST,), jnp.int32),
        pltpu.VMEM((_COPIES * _HIST,), jnp.float32),
        pltpu.VMEM((512,), jnp.int32),
        pltpu.VMEM((512,), jnp.float32),
        pltpu.VMEM((2, 512), jnp.int32),
        pltpu.VMEM((2, 512), jnp.float32),
        pltpu.VMEM((16, 512), jnp.int32),
        pltpu.VMEM((16, 512), jnp.float32),
        pltpu.VMEM((16,), jnp.int32),
        pltpu.VMEM((16,), jnp.float32),
        pltpu.VMEM_SHARED((16, 512), jnp.int32),
        pltpu.VMEM_SHARED((16, 512), jnp.float32),
        pltpu.SemaphoreType.DMA,
        pltpu.SemaphoreType.DMA,
    ]

    zc = jnp.zeros((_NCORE, 512), jnp.int32)
    zs = jnp.zeros((_NCORE, 512), jnp.float32)
    zi = jnp.zeros((_NCORE, 16), jnp.int32)
    zf = jnp.zeros((_NCORE, 16), jnp.float32)

    hc, hs = None, None
    sti, stf = zi, zf
    for l in range(4):
        f = pl.kernel(
            functools.partial(_scan_body, l),
            out_type=hist_out + st_out,
            mesh=mesh,
            compiler_params=params,
            scratch_types=scan_scratch,
        )
        hc, hs, sti_n, stf_n = f(keys,
                                 zc if hc is None else hc,
                                 zs if hs is None else hs,
                                 sti, stf)
        if l > 0:
            sti, stf = sti_n, stf_n

    f = pl.kernel(
        _final_body,
        out_type=jax.ShapeDtypeStruct((16,), jnp.float32),
        mesh=mesh,
        compiler_params=params,
        scratch_types=[
            pltpu.VMEM((512,), jnp.int32),
            pltpu.VMEM((512,), jnp.float32),
            pltpu.VMEM((2, 512), jnp.int32),
            pltpu.VMEM((2, 512), jnp.float32),
            pltpu.VMEM((16,), jnp.int32),
            pltpu.VMEM((16,), jnp.float32),
            pltpu.VMEM((16,), jnp.float32),
        ],
    )
    return f(hc, hs, sti, stf)


def kernel(inputs, targets):
    keys = _compute_keys(inputs, targets)
    out = _sc_select(keys)
    return out[0]
